# baseline (speedup 1.0000x reference)
import dataclasses
import functools

import jax
import jax.numpy as jnp
import numpy as np
from jax import lax
from jax.experimental import pallas as pl
from jax.experimental.pallas import tpu as pltpu
from jax.experimental.pallas import tpu_sc as plsc

N = 4096
B = 256
NNZ = 167772
L = 16
NC = 2
NS = 16
W = 128
SB = 2
MD = 4
NSB = -(-NNZ // (NC * SB * W))
NWIN = NSB * SB
NSB_PAD = NSB + MD
ENT_PAD = NC * NSB_PAD * SB * W

_sc_mesh = plsc.VectorSubcoreMesh(core_axis_name="c", subcore_axis_name="s")

_sc_params = pltpu.CompilerParams()
if "needs_layout_passes" in pltpu.CompilerParams.__dataclass_fields__:
    _sc_params = dataclasses.replace(_sc_params, needs_layout_passes=False)
if "use_tc_tiling_on_sc" in pltpu.CompilerParams.__dataclass_fields__:
    _sc_params = dataclasses.replace(_sc_params, use_tc_tiling_on_sc=False)


def _spmm_body(x_hbm, c16_hbm, v16_hbm, eidx_hbm, z_hbm, yt_hbm,
               mc, mv, me, cw2b, gb, yt, msem, gsem):
    h = lax.axis_index("c")
    s = lax.axis_index("s")
    svec = jnp.full((L,), s, jnp.int32)

    pltpu.sync_copy(z_hbm, yt)

    def issue_meta(sb, b):
        pltpu.async_copy(c16_hbm.at[h, sb], mc.at[b], msem.at[b, 0])
        pltpu.async_copy(v16_hbm.at[h, sb], mv.at[b], msem.at[b, 1])
        pltpu.async_copy(eidx_hbm.at[h, sb], me.at[b], msem.at[b, 2])

    def wait_meta(sb, b):
        pltpu.make_async_copy(c16_hbm.at[h, sb], mc.at[b], msem.at[b, 0]).wait()
        pltpu.make_async_copy(v16_hbm.at[h, sb], mv.at[b], msem.at[b, 1]).wait()
        pltpu.make_async_copy(eidx_hbm.at[h, sb], me.at[b], msem.at[b, 2]).wait()

    def issue_gathers(b):
        for wi in range(SB):
            k = b * SB + wi
            for grp in range(W // L):
                sl = pl.ds(grp * L, L)
                cw2b[k, sl] = mc[b, wi, sl] + svec
            pltpu.async_copy(x_hbm.at[cw2b.at[k]], gb.at[k], gsem.at[k])

    def wait_gather(k):
        pltpu.make_async_copy(x_hbm.at[cw2b.at[k]], gb.at[k], gsem.at[k]).wait()

    def compute(b, wi):
        k = b * SB + wi
        wait_gather(k)

        def _ent(e):
            prod = gb[k, e, pl.ds(0, L)] * mv[b, wi, e, pl.ds(0, L)]
            plsc.addupdate_scatter(yt, [me[b, wi, e, pl.ds(0, L)]], prod)
        plsc.parallel_loop(0, W, unroll=8)(_ent)

    for q in range(MD - 1):
        issue_meta(q, q)
    for q in range(MD - 1):
        wait_meta(q, q)
        issue_gathers(q)
    issue_meta(MD - 1, MD - 1)

    @pl.loop(0, NSB, step=MD)
    def _sb(t):
        for q in range(MD):
            sb = t + q
            q3 = (q + MD - 1) % MD
            compute(q, 0)
            wait_meta(sb + MD - 1, q3)
            issue_gathers(q3)
            for wi in range(1, SB):
                compute(q, wi)
            issue_meta(sb + MD, q)

    wait_meta(NSB + MD - 1, MD - 1)
    for q in range(MD - 1):
        for wi in range(SB):
            wait_gather(q * SB + wi)

    pltpu.sync_copy(yt, yt_hbm.at[h, s])


@functools.partial(
    pl.kernel,
    out_type=jax.ShapeDtypeStruct((NC, NS, NS * N), jnp.float32),
    mesh=_sc_mesh,
    scratch_types=[
        pltpu.VMEM((MD, SB, W), jnp.int32),
        pltpu.VMEM((MD, SB, W, L), jnp.float32),
        pltpu.VMEM((MD, SB, W, L), jnp.int32),
        pltpu.VMEM((MD * SB, W), jnp.int32),
        pltpu.VMEM((MD * SB, W, L), jnp.float32),
        pltpu.VMEM((NS * N,), jnp.float32),
        pltpu.SemaphoreType.DMA((MD, 3)),
        pltpu.SemaphoreType.DMA((MD * SB,)),
    ],
    compiler_params=_sc_params,
)
def _spmm_sc(x_hbm, c16_hbm, v16_hbm, eidx_hbm, z_hbm, yt_hbm,
             mc, mv, me, cw2b, gb, yt, msem, gsem):
    _spmm_body(x_hbm, c16_hbm, v16_hbm, eidx_hbm, z_hbm, yt_hbm,
               mc, mv, me, cw2b, gb, yt, msem, gsem)


def _addT_body(ya_ref, yb_ref, o_ref):
    o_ref[...] = (ya_ref[...] + yb_ref[...]).T


def _addT(yt2):
    return pl.pallas_call(
        _addT_body,
        out_shape=jax.ShapeDtypeStruct((N, B), jnp.float32),
    )(yt2[0], yt2[1])


def _final_body(ya_ref, yb_ref, bias_ref, o_ref):
    o_ref[...] = ya_ref[...] + yb_ref[...] + bias_ref[...]


def _final(yt2, bias):
    return pl.pallas_call(
        _final_body,
        out_shape=jax.ShapeDtypeStruct((B, N), jnp.float32),
    )(yt2[0], yt2[1], bias.reshape(1, N))


_LIOTA = np.arange(L, dtype=np.int32) * N


def _prep(vals, rows, cols):
    ent_real = NC * NSB * SB * W
    pad = ent_real - NNZ
    sb_pad = ((0, 0), (0, NSB_PAD - NSB), (0, 0), (0, 0), (0, 0))
    v = jnp.pad(vals, (0, pad))
    r = jnp.pad(rows.astype(jnp.int32), (0, pad))
    c = jnp.pad(cols.astype(jnp.int32), (0, pad))
    c16 = jnp.pad((c * L).reshape(NC, NSB, SB, W, 1), sb_pad)[..., 0]
    v16 = jnp.pad(jnp.broadcast_to(v[:, None], (ent_real, L)).reshape(
        NC, NSB, SB, W, L), sb_pad)
    eidx = jnp.pad((r[:, None] + jnp.asarray(_LIOTA)[None, :]).reshape(
        NC, NSB, SB, W, L), sb_pad)
    return c16, v16, eidx


def kernel(U, vals0, rows0, cols0, vals1, rows1, cols1, vals2, rows2, cols2, bias):
    zeros = jnp.zeros((NS * N,), jnp.float32)
    x = U.T
    for vals, rows, cols in ((vals2, rows2, cols2),
                             (vals1, rows1, cols1)):
        c16, v16, eidx = _prep(vals, rows, cols)
        yt2 = _spmm_sc(x.reshape(N * L, L), c16, v16, eidx, zeros)
        yt2 = yt2.reshape(NC, B, N)
        x = _addT(yt2)
    c16, v16, eidx = _prep(vals0, rows0, cols0)
    yt2 = _spmm_sc(x.reshape(N * L, L), c16, v16, eidx, zeros)
    return _final(yt2.reshape(NC, B, N), bias)


# speedup vs baseline: 1.2122x; 1.2122x over previous
import dataclasses
import functools

import jax
import jax.numpy as jnp
from jax.lax import GatherDimensionNumbers, GatherScatterMode
import numpy as np
from jax import lax
from jax.experimental import pallas as pl
from jax.experimental.pallas import tpu as pltpu
from jax.experimental.pallas import tpu_sc as plsc

N = 4096
B = 256
NNZ = 167772
L = 16
NC = 2
NS = 16
W = 128
SB = 4
NSB = -(-NNZ // (NC * SB * W))
NWIN = NSB * SB
NSB_PAD = NSB + 2
ENT_PAD = NC * NSB_PAD * SB * W

_sc_mesh = plsc.VectorSubcoreMesh(core_axis_name="c", subcore_axis_name="s")

_sc_params = pltpu.CompilerParams()
if "needs_layout_passes" in pltpu.CompilerParams.__dataclass_fields__:
    _sc_params = dataclasses.replace(_sc_params, needs_layout_passes=False)
if "use_tc_tiling_on_sc" in pltpu.CompilerParams.__dataclass_fields__:
    _sc_params = dataclasses.replace(_sc_params, use_tc_tiling_on_sc=False)


def _spmm_body(x_hbm, c16_hbm, v16_hbm, eidx_hbm, z_hbm, yt_hbm,
               mc, mv, me, cw2b, gb, yt, msem, gsem):
    h = lax.axis_index("c")
    s = lax.axis_index("s")
    svec = jnp.full((L,), s, jnp.int32)
    liota = lax.iota(jnp.int32, L) * N

    def bcast(vec, d):
        return lax.gather(
            vec, jnp.full((L, 1), d, jnp.int32),
            GatherDimensionNumbers(offset_dims=(), collapsed_slice_dims=(0,),
                                   start_index_map=(0,)),
            (1,), mode=GatherScatterMode.PROMISE_IN_BOUNDS)

    pltpu.sync_copy(z_hbm, yt)

    def issue_meta(sb, b):
        pltpu.async_copy(c16_hbm.at[h, sb], mc.at[b], msem.at[b, 0])
        pltpu.async_copy(v16_hbm.at[h, sb], mv.at[b], msem.at[b, 1])
        pltpu.async_copy(eidx_hbm.at[h, sb], me.at[b], msem.at[b, 2])

    def wait_meta(sb, b):
        pltpu.make_async_copy(c16_hbm.at[h, sb], mc.at[b], msem.at[b, 0]).wait()
        pltpu.make_async_copy(v16_hbm.at[h, sb], mv.at[b], msem.at[b, 1]).wait()
        pltpu.make_async_copy(eidx_hbm.at[h, sb], me.at[b], msem.at[b, 2]).wait()

    def issue_gathers(b):
        for wi in range(SB):
            k = b * SB + wi
            for grp in range(W // L):
                sl = pl.ds(grp * L, L)
                cw2b[k, sl] = mc[b, wi, sl] + svec
            pltpu.async_copy(x_hbm.at[cw2b.at[k]], gb.at[k], gsem.at[k])

    def wait_gather(k):
        pltpu.make_async_copy(x_hbm.at[cw2b.at[k]], gb.at[k], gsem.at[k]).wait()

    def compute(b, wi):
        k = b * SB + wi
        wait_gather(k)

        def _grp(e0):
            vv = mv[b, wi, pl.ds(e0, L)]
            rv = me[b, wi, pl.ds(e0, L)]
            for d in range(L):
                e = e0 + d
                prod = gb[k, e, pl.ds(0, L)] * bcast(vv, d)
                plsc.addupdate_scatter(yt, [bcast(rv, d) + liota], prod)
        plsc.parallel_loop(0, W, step=L, unroll=2)(_grp)

    issue_meta(0, 0)
    wait_meta(0, 0)
    issue_gathers(0)
    issue_meta(1, 1)

    @pl.loop(0, NSB, step=2)
    def _sb(t):
        for b in range(2):
            sb = t + b
            compute(b, 0)
            wait_meta(sb + 1, 1 - b)
            issue_gathers(1 - b)
            for wi in range(1, SB):
                compute(b, wi)
            issue_meta(sb + 2, b)

    wait_meta(NSB + 1, 1)
    for k in range(SB):
        wait_gather(k)

    pltpu.sync_copy(yt, yt_hbm.at[h, s])


@functools.partial(
    pl.kernel,
    out_type=jax.ShapeDtypeStruct((NC, NS, NS * N), jnp.float32),
    mesh=_sc_mesh,
    scratch_types=[
        pltpu.VMEM((2, SB, W), jnp.int32),
        pltpu.VMEM((2, SB, W), jnp.float32),
        pltpu.VMEM((2, SB, W), jnp.int32),
        pltpu.VMEM((2 * SB, W), jnp.int32),
        pltpu.VMEM((2 * SB, W, L), jnp.float32),
        pltpu.VMEM((NS * N,), jnp.float32),
        pltpu.SemaphoreType.DMA((2, 3)),
        pltpu.SemaphoreType.DMA((2 * SB,)),
    ],
    compiler_params=_sc_params,
)
def _spmm_sc(x_hbm, c16_hbm, v16_hbm, eidx_hbm, z_hbm, yt_hbm,
             mc, mv, me, cw2b, gb, yt, msem, gsem):
    _spmm_body(x_hbm, c16_hbm, v16_hbm, eidx_hbm, z_hbm, yt_hbm,
               mc, mv, me, cw2b, gb, yt, msem, gsem)


def _addT_body(ya_ref, yb_ref, o_ref):
    o_ref[...] = (ya_ref[...] + yb_ref[...]).T


def _addT(yt2):
    return pl.pallas_call(
        _addT_body,
        out_shape=jax.ShapeDtypeStruct((N, B), jnp.float32),
    )(yt2[0], yt2[1])


def _final_body(ya_ref, yb_ref, bias_ref, o_ref):
    o_ref[...] = ya_ref[...] + yb_ref[...] + bias_ref[...]


def _final(yt2, bias):
    return pl.pallas_call(
        _final_body,
        out_shape=jax.ShapeDtypeStruct((B, N), jnp.float32),
    )(yt2[0], yt2[1], bias.reshape(1, N))


_LIOTA = np.arange(L, dtype=np.int32) * N


def _prep(vals, rows, cols):
    ent_real = NC * NSB * SB * W
    pad = ent_real - NNZ
    sb_pad = ((0, 0), (0, NSB_PAD - NSB), (0, 0), (0, 0), (0, 0))
    v = jnp.pad(vals, (0, pad))
    r = jnp.pad(rows.astype(jnp.int32), (0, pad))
    c = jnp.pad(cols.astype(jnp.int32), (0, pad))
    sb_pad4 = sb_pad[:-1]
    c16 = jnp.pad((c * L).reshape(NC, NSB, SB, W), sb_pad4)
    vc = jnp.pad(v.reshape(NC, NSB, SB, W), sb_pad4)
    rc = jnp.pad(r.reshape(NC, NSB, SB, W), sb_pad4)
    return c16, vc, rc


def kernel(U, vals0, rows0, cols0, vals1, rows1, cols1, vals2, rows2, cols2, bias):
    zeros = jnp.zeros((NS * N,), jnp.float32)
    x = U.T
    for vals, rows, cols in ((vals2, rows2, cols2),
                             (vals1, rows1, cols1)):
        c16, v16, eidx = _prep(vals, rows, cols)
        yt2 = _spmm_sc(x.reshape(N * L, L), c16, v16, eidx, zeros)
        yt2 = yt2.reshape(NC, B, N)
        x = _addT(yt2)
    c16, v16, eidx = _prep(vals0, rows0, cols0)
    yt2 = _spmm_sc(x.reshape(N * L, L), c16, v16, eidx, zeros)
    return _final(yt2.reshape(NC, B, N), bias)


# speedup vs baseline: 1.2202x; 1.0066x over previous
import dataclasses
import functools

import jax
import jax.numpy as jnp
from jax.lax import GatherDimensionNumbers, GatherScatterMode
import numpy as np
from jax import lax
from jax.experimental import pallas as pl
from jax.experimental.pallas import tpu as pltpu
from jax.experimental.pallas import tpu_sc as plsc

N = 4096
B = 256
NNZ = 167772
L = 16
NC = 2
NS = 16
W = 128
SB = 4
NSB = -(-NNZ // (NC * SB * W))
NWIN = NSB * SB
NSB_PAD = NSB + 2
ENT_PAD = NC * NSB_PAD * SB * W

_sc_mesh = plsc.VectorSubcoreMesh(core_axis_name="c", subcore_axis_name="s")

_sc_params = pltpu.CompilerParams()
if "needs_layout_passes" in pltpu.CompilerParams.__dataclass_fields__:
    _sc_params = dataclasses.replace(_sc_params, needs_layout_passes=False)
if "use_tc_tiling_on_sc" in pltpu.CompilerParams.__dataclass_fields__:
    _sc_params = dataclasses.replace(_sc_params, use_tc_tiling_on_sc=False)


def _spmm_body(x_hbm, c16_hbm, v16_hbm, eidx_hbm, z_hbm, yt_hbm,
               mc, mv, me, cw2b, gb, yt, msem, gsem):
    h = lax.axis_index("c")
    s = lax.axis_index("s")
    svec = jnp.full((L,), s, jnp.int32)
    liota = lax.iota(jnp.int32, L) * N

    def bcast(vec, d):
        return lax.gather(
            vec, jnp.full((L, 1), d, jnp.int32),
            GatherDimensionNumbers(offset_dims=(), collapsed_slice_dims=(0,),
                                   start_index_map=(0,)),
            (1,), mode=GatherScatterMode.PROMISE_IN_BOUNDS)

    pltpu.sync_copy(z_hbm, yt)

    def issue_meta(sb, b):
        pltpu.async_copy(c16_hbm.at[h, sb], mc.at[b], msem.at[b, 0])
        pltpu.async_copy(v16_hbm.at[h, sb], mv.at[b], msem.at[b, 1])
        pltpu.async_copy(eidx_hbm.at[h, sb], me.at[b], msem.at[b, 2])

    def wait_meta(sb, b):
        pltpu.make_async_copy(c16_hbm.at[h, sb], mc.at[b], msem.at[b, 0]).wait()
        pltpu.make_async_copy(v16_hbm.at[h, sb], mv.at[b], msem.at[b, 1]).wait()
        pltpu.make_async_copy(eidx_hbm.at[h, sb], me.at[b], msem.at[b, 2]).wait()

    def issue_gathers(b):
        for wi in range(SB):
            k = b * SB + wi
            for grp in range(W // L):
                sl = pl.ds(grp * L, L)
                cw2b[k, sl] = mc[b, wi, sl] + svec
            pltpu.async_copy(x_hbm.at[cw2b.at[k]], gb.at[k], gsem.at[k])

    def wait_gather(k):
        pltpu.make_async_copy(x_hbm.at[cw2b.at[k]], gb.at[k], gsem.at[k]).wait()

    def compute(b, wi):
        k = b * SB + wi
        wait_gather(k)

        def _grp(e0):
            vv = mv[b, wi, pl.ds(e0, L)]
            rv = me[b, wi, pl.ds(e0, L)]
            for d in range(L):
                e = e0 + d
                prod = gb[k, e, pl.ds(0, L)] * bcast(vv, d)
                plsc.addupdate_scatter(yt, [bcast(rv, d) + liota], prod)
        plsc.parallel_loop(0, W, step=L, unroll=4)(_grp)

    issue_meta(0, 0)
    wait_meta(0, 0)
    issue_gathers(0)
    issue_meta(1, 1)

    @pl.loop(0, NSB, step=2)
    def _sb(t):
        for b in range(2):
            sb = t + b
            compute(b, 0)
            wait_meta(sb + 1, 1 - b)
            issue_gathers(1 - b)
            for wi in range(1, SB):
                compute(b, wi)
            issue_meta(sb + 2, b)

    wait_meta(NSB + 1, 1)
    for k in range(SB):
        wait_gather(k)

    pltpu.sync_copy(yt, yt_hbm.at[h, s])


@functools.partial(
    pl.kernel,
    out_type=jax.ShapeDtypeStruct((NC, NS, NS * N), jnp.float32),
    mesh=_sc_mesh,
    scratch_types=[
        pltpu.VMEM((2, SB, W), jnp.int32),
        pltpu.VMEM((2, SB, W), jnp.float32),
        pltpu.VMEM((2, SB, W), jnp.int32),
        pltpu.VMEM((2 * SB, W), jnp.int32),
        pltpu.VMEM((2 * SB, W, L), jnp.float32),
        pltpu.VMEM((NS * N,), jnp.float32),
        pltpu.SemaphoreType.DMA((2, 3)),
        pltpu.SemaphoreType.DMA((2 * SB,)),
    ],
    compiler_params=_sc_params,
)
def _spmm_sc(x_hbm, c16_hbm, v16_hbm, eidx_hbm, z_hbm, yt_hbm,
             mc, mv, me, cw2b, gb, yt, msem, gsem):
    _spmm_body(x_hbm, c16_hbm, v16_hbm, eidx_hbm, z_hbm, yt_hbm,
               mc, mv, me, cw2b, gb, yt, msem, gsem)


def _addT_body(ya_ref, yb_ref, o_ref):
    o_ref[...] = (ya_ref[...] + yb_ref[...]).T


def _addT(yt2):
    return pl.pallas_call(
        _addT_body,
        out_shape=jax.ShapeDtypeStruct((N, B), jnp.float32),
    )(yt2[0], yt2[1])


def _final_body(ya_ref, yb_ref, bias_ref, o_ref):
    o_ref[...] = ya_ref[...] + yb_ref[...] + bias_ref[...]


def _final(yt2, bias):
    return pl.pallas_call(
        _final_body,
        out_shape=jax.ShapeDtypeStruct((B, N), jnp.float32),
    )(yt2[0], yt2[1], bias.reshape(1, N))


_LIOTA = np.arange(L, dtype=np.int32) * N


def _prep(vals, rows, cols):
    ent_real = NC * NSB * SB * W
    pad = ent_real - NNZ
    sb_pad = ((0, 0), (0, NSB_PAD - NSB), (0, 0), (0, 0), (0, 0))
    v = jnp.pad(vals, (0, pad))
    r = jnp.pad(rows.astype(jnp.int32), (0, pad))
    c = jnp.pad(cols.astype(jnp.int32), (0, pad))
    sb_pad4 = sb_pad[:-1]
    c16 = jnp.pad((c * L).reshape(NC, NSB, SB, W), sb_pad4)
    vc = jnp.pad(v.reshape(NC, NSB, SB, W), sb_pad4)
    rc = jnp.pad(r.reshape(NC, NSB, SB, W), sb_pad4)
    return c16, vc, rc


def kernel(U, vals0, rows0, cols0, vals1, rows1, cols1, vals2, rows2, cols2, bias):
    zeros = jnp.zeros((NS * N,), jnp.float32)
    x = U.T
    for vals, rows, cols in ((vals2, rows2, cols2),
                             (vals1, rows1, cols1)):
        c16, v16, eidx = _prep(vals, rows, cols)
        yt2 = _spmm_sc(x.reshape(N * L, L), c16, v16, eidx, zeros)
        yt2 = yt2.reshape(NC, B, N)
        x = _addT(yt2)
    c16, v16, eidx = _prep(vals0, rows0, cols0)
    yt2 = _spmm_sc(x.reshape(N * L, L), c16, v16, eidx, zeros)
    return _final(yt2.reshape(NC, B, N), bias)
